# Initial kernel scaffold; baseline (speedup 1.0000x reference)
import jax
import jax.numpy as jnp
from jax.experimental import pallas as pl


def kernel(x, down_W, down_b, up_W, up_b, cW, cb, tW, tb, dw_W, dw_g, dw_be, pw_W, pw_g, pw_be):
    raise NotImplementedError("write your pallas kernel here")


# speedup vs baseline: 1.6668x; 1.6668x over previous
import functools

import jax
import jax.numpy as jnp
from jax import lax
from jax.experimental import pallas as pl

HD = 64
PREC = lax.Precision.DEFAULT
F32 = jnp.float32


def _dot(a, b, dims):
    return lax.dot_general(a, b, (dims, ((), ())), precision=PREC,
                           preferred_element_type=F32)


def _mm(a, b):
    return _dot(a, b, (((a.ndim - 1,), (0,))))


def _down_body(x_ref, w_ref, b_ref, o_ref):
    t = pl.program_id(1)
    acc = _mm(w_ref[0], x_ref[0, 0])

    @pl.when(t == 0)
    def _():
        o_ref[0] = acc

    @pl.when(t > 0)
    def _():
        o_ref[0] = o_ref[0] + acc

    @pl.when(t == 15)
    def _():
        o_ref[0] = o_ref[0] + b_ref[...]


def _down_conv(x, W, b):
    B, C, H, Wd = x.shape
    h, w = H // 2, Wd // 2
    N = h * w
    xp = jnp.pad(x, ((0, 0), (0, 0), (1, 1), (1, 1)))
    taps = [xp[:, :, kh:kh + 2 * h - 1:2, kw:kw + 2 * w - 1:2].reshape(B, C, N)
            for kh in range(4) for kw in range(4)]
    X = jnp.stack(taps, axis=1)
    Wt = W.transpose(2, 3, 0, 1).reshape(16, C, C)
    b2 = b.reshape(C, 1)
    return pl.pallas_call(
        _down_body,
        grid=(B, 16),
        in_specs=[
            pl.BlockSpec((1, 1, C, N), lambda b_, t: (b_, t, 0, 0)),
            pl.BlockSpec((1, C, C), lambda b_, t: (t, 0, 0)),
            pl.BlockSpec((C, 1), lambda b_, t: (0, 0)),
        ],
        out_specs=pl.BlockSpec((1, C, N), lambda b_, t: (b_, 0, 0)),
        out_shape=jax.ShapeDtypeStruct((B, C, N), F32),
    )(X, Wt, b2)


def _coarse_body(kf, xd_ref, w_ref, b_ref, o_ref, sel_ref):
    tok = xd_ref[0]
    N = tok.shape[1]
    scale = HD ** (-0.5)
    t = tok.T
    qkv = _mm(t, w_ref[...]) + b_ref[...]
    q = qkv[:, :HD]
    k = qkv[:, HD:2 * HD]
    v = qkv[:, 2 * HD:]
    ls = _dot(q, k, ((1,), (1,))) * scale
    m = jnp.max(ls, axis=1, keepdims=True)
    p = jnp.exp(ls - m)
    s = jnp.sum(p, axis=1, keepdims=True)
    attn = p / s
    score = jnp.sum(attn, axis=0, keepdims=True)
    out = _mm(attn, v)
    o_ref[0] = out.T

    bits = lax.bitcast_convert_type(score, jnp.int32)

    def body(i, th):
        cand = th | lax.shift_left(jnp.int32(1), 30 - i)
        cnt = jnp.sum((bits >= cand).astype(jnp.int32))
        return jnp.where(cnt >= kf, cand, th)

    th = lax.fori_loop(0, 31, body, jnp.int32(0))
    gt = bits > th
    eq = bits == th
    n_gt = jnp.sum(gt.astype(jnp.int32))
    deficit = (kf - n_gt).astype(F32)
    r = lax.broadcasted_iota(jnp.int32, (N, N), 0)
    c = lax.broadcasted_iota(jnp.int32, (N, N), 1)
    ut = (r <= c).astype(F32)
    cum_eq = _mm(eq.astype(F32), ut)
    mask = jnp.logical_or(gt, jnp.logical_and(eq, cum_eq <= deficit))
    slot = (_mm(mask.astype(F32), ut) - 1.0).astype(jnp.int32)
    si = lax.broadcasted_iota(jnp.int32, (N, kf), 1)
    oh = jnp.logical_and(slot.T == si, mask.T).astype(F32)
    sel_ref[0, 0] = oh


def _coarse_attn(xd, cW, cb, kf):
    B, C, N = xd.shape
    nh = C // HD
    cb2 = cb.reshape(1, 3 * HD)
    return pl.pallas_call(
        functools.partial(_coarse_body, kf),
        grid=(B, nh),
        in_specs=[
            pl.BlockSpec((1, HD, N), lambda b_, h_: (b_, h_, 0)),
            pl.BlockSpec((HD, 3 * HD), lambda b_, h_: (0, 0)),
            pl.BlockSpec((1, 3 * HD), lambda b_, h_: (0, 0)),
        ],
        out_specs=(
            pl.BlockSpec((1, HD, N), lambda b_, h_: (b_, h_, 0)),
            pl.BlockSpec((1, 1, N, kf), lambda b_, h_: (b_, h_, 0, 0)),
        ),
        out_shape=(
            jax.ShapeDtypeStruct((B, C, N), F32),
            jax.ShapeDtypeStruct((B, nh, N, kf), F32),
        ),
    )(xd, cW, cb2)


def _up_body(x_ref, w_ref, b_ref, o_ref):
    j = pl.program_id(2)
    acc = _mm(w_ref[0, 0], x_ref[0, 0])

    @pl.when(j == 0)
    def _():
        o_ref[0, 0] = acc

    @pl.when(j > 0)
    def _():
        o_ref[0, 0] = o_ref[0, 0] + acc

    @pl.when(j == 3)
    def _():
        o_ref[0, 0] = o_ref[0, 0] + b_ref[...]


def _up_conv(x, W, b):
    B, C, h, w = x.shape
    N = h * w
    xp = jnp.pad(x, ((0, 0), (0, 0), (1, 1), (1, 1)))
    U = jnp.stack([xp[:, :, sh + 1:sh + 1 + h, sw + 1:sw + 1 + w].reshape(B, C, N)
                   for sh in (-1, 0, 1) for sw in (-1, 0, 1)], axis=1)
    Wt = W.transpose(2, 3, 1, 0)
    Wup = jnp.stack([
        jnp.stack([Wt[2 * jh + (1 - ei), 2 * jw + (1 - ej)]
                   for jh in range(2) for jw in range(2)], axis=0)
        for ei in range(2) for ej in range(2)], axis=0)
    b2 = b.reshape(C, 1)

    def xmap(b_, f, j):
        ei, ej = f // 2, f % 2
        jh, jw = j // 2, j % 2
        return (b_, (ei - jh + 1) * 3 + (ej - jw + 1), 0, 0)

    return pl.pallas_call(
        _up_body,
        grid=(B, 4, 4),
        in_specs=[
            pl.BlockSpec((1, 1, C, N), xmap),
            pl.BlockSpec((1, 1, C, C), lambda b_, f, j: (f, j, 0, 0)),
            pl.BlockSpec((C, 1), lambda b_, f, j: (0, 0)),
        ],
        out_specs=pl.BlockSpec((1, 1, C, N), lambda b_, f, j: (b_, f, 0, 0)),
        out_shape=jax.ShapeDtypeStruct((B, 4, C, N), F32),
    )(U, Wup, b2)


def _topk_body(tok_ref, sel_ref, w_ref, b_ref, o_ref):
    sel = sel_ref[0, 0]
    tokg = tok_ref[0, 0]
    kf = sel.shape[1]
    scale = HD ** (-0.5)
    tsel = _dot(sel, tokg, ((0,), (0,)))
    toks = jnp.concatenate([tsel[:, f * HD:(f + 1) * HD] for f in range(4)],
                           axis=0)
    qkv = _mm(toks, w_ref[...]) + b_ref[...]
    q = qkv[:, :HD]
    k = qkv[:, HD:2 * HD]
    v = qkv[:, 2 * HD:]
    ls = _dot(q, k, ((1,), (1,))) * scale
    m = jnp.max(ls, axis=1, keepdims=True)
    p = jnp.exp(ls - m)
    s = jnp.sum(p, axis=1, keepdims=True)
    out = _mm(p / s, v)
    og = jnp.concatenate([out[f * kf:(f + 1) * kf, :] for f in range(4)],
                         axis=1)
    o_ref[0, 0] = _mm(sel, og)


def _topk_attn(tokg, sel, tW, tb):
    B, nh, P, D4 = tokg.shape
    kf = sel.shape[3]
    tb2 = tb.reshape(1, 3 * HD)
    return pl.pallas_call(
        _topk_body,
        grid=(B, nh),
        in_specs=[
            pl.BlockSpec((1, 1, P, D4), lambda b_, h_: (b_, h_, 0, 0)),
            pl.BlockSpec((1, 1, P, kf), lambda b_, h_: (b_, h_, 0, 0)),
            pl.BlockSpec((HD, 3 * HD), lambda b_, h_: (0, 0)),
            pl.BlockSpec((1, 3 * HD), lambda b_, h_: (0, 0)),
        ],
        out_specs=pl.BlockSpec((1, 1, P, D4), lambda b_, h_: (b_, h_, 0, 0)),
        out_shape=jax.ShapeDtypeStruct((B, nh, P, D4), F32),
    )(tokg, sel, tW, tb2)


def _dw_body(x_ref, w_ref, g_ref, be_ref, o_ref):
    xp = x_ref[0]
    H = xp.shape[1] - 2
    W = xp.shape[2] - 2
    y = jnp.zeros((xp.shape[0], H, W), F32)
    for u in range(3):
        for v in range(3):
            y = y + w_ref[u * 3 + v] * xp[:, u:u + H, v:v + W]
    y = y * g_ref[...] + be_ref[...]
    o_ref[0] = jnp.clip(y, 0.0, 6.0)


def _dw_conv(x2, dw_W, dw_g, dw_be):
    B, C, H, W = x2.shape
    cb = 128 if C % 128 == 0 else C
    nc = C // cb
    xp = jnp.pad(x2, ((0, 0), (0, 0), (1, 1), (1, 1)))
    dwr = dw_W.reshape(C, 9).T.reshape(9, C, 1, 1)
    g = dw_g.reshape(C, 1, 1)
    be = dw_be.reshape(C, 1, 1)
    return pl.pallas_call(
        _dw_body,
        grid=(B, nc),
        in_specs=[
            pl.BlockSpec((1, cb, H + 2, W + 2), lambda b_, c_: (b_, c_, 0, 0)),
            pl.BlockSpec((9, cb, 1, 1), lambda b_, c_: (0, c_, 0, 0)),
            pl.BlockSpec((cb, 1, 1), lambda b_, c_: (c_, 0, 0)),
            pl.BlockSpec((cb, 1, 1), lambda b_, c_: (c_, 0, 0)),
        ],
        out_specs=pl.BlockSpec((1, cb, H, W), lambda b_, c_: (b_, c_, 0, 0)),
        out_shape=jax.ShapeDtypeStruct((B, C, H, W), F32),
    )(xp, dwr, g, be)


def _pw_body(x_ref, w_ref, g_ref, be_ref, o_ref):
    z = _mm(w_ref[...], x_ref[0])
    z = z * g_ref[...] + be_ref[...]
    o_ref[0] = jnp.clip(z, 0.0, 6.0)


def _pw_conv(t, pw_W, pw_g, pw_be):
    B, C, N = t.shape
    nb = 1
    nn = N
    W2 = pw_W.reshape(C, C)
    g = pw_g.reshape(C, 1)
    be = pw_be.reshape(C, 1)
    return pl.pallas_call(
        _pw_body,
        grid=(B, nb),
        in_specs=[
            pl.BlockSpec((1, C, nn), lambda b_, n_: (b_, 0, n_)),
            pl.BlockSpec((C, C), lambda b_, n_: (0, 0)),
            pl.BlockSpec((C, 1), lambda b_, n_: (0, 0)),
            pl.BlockSpec((C, 1), lambda b_, n_: (0, 0)),
        ],
        out_specs=pl.BlockSpec((1, C, nn), lambda b_, n_: (b_, 0, n_)),
        out_shape=jax.ShapeDtypeStruct((B, C, N), F32),
    )(t, W2, g, be)


def kernel(x, down_W, down_b, up_W, up_b, cW, cb, tW, tb,
           dw_W, dw_g, dw_be, pw_W, pw_g, pw_be):
    B, C, H, W = x.shape
    nh = C // HD
    h, w = H // 2, W // 2
    N = h * w
    kf = max(1, N // 4)

    xd = _down_conv(x, down_W, down_b)
    att, sel = _coarse_attn(xd, cW, cb, kf)
    co_ph = _up_conv(att.reshape(B, C, h, w), up_W, up_b)

    tokg = (co_ph.reshape(B, 4, nh, HD, N)
            .transpose(0, 2, 4, 1, 3).reshape(B, nh, N, 4 * HD))
    scat = _topk_attn(tokg, sel, tW, tb)
    res_ph = (scat.reshape(B, nh, N, 4, HD)
              .transpose(0, 3, 1, 4, 2).reshape(B, 4, C, N))
    x2_ph = co_ph + res_ph
    x2 = (x2_ph.reshape(B, 2, 2, C, h, w)
          .transpose(0, 3, 4, 1, 5, 2).reshape(B, C, H, W))

    t = _dw_conv(x2, dw_W, dw_g, dw_be)
    z = _pw_conv(t.reshape(B, C, H * W), pw_W, pw_g, pw_be)
    return z.reshape(B, C, H, W)


# speedup vs baseline: 3.7542x; 2.2523x over previous
import functools

import jax
import jax.numpy as jnp
from jax import lax
from jax.experimental import pallas as pl
from jax.experimental.pallas import tpu as pltpu

HD = 64
PREC = lax.Precision.DEFAULT
F32 = jnp.float32


def _dot(a, b, dims):
    return lax.dot_general(a, b, (dims, ((), ())), precision=PREC,
                           preferred_element_type=F32)


def _mm(a, b):
    return _dot(a, b, (((a.ndim - 1,), (0,))))


def _shift_mask(x, m, n, w, nrow):
    N = nrow * w
    d = m * w + n
    sh = pltpu.roll(x, (-d) % N, axis=1)
    li = lax.broadcasted_iota(jnp.int32, (1, N), 1)
    col = li - (li // w) * w
    src = li + d
    valid = jnp.logical_and(
        jnp.logical_and(src >= 0, src < N),
        jnp.logical_and(col + n >= 0, col + n < w))
    return jnp.where(valid, sh, 0.0)


def _down_body(w_, nrow, x_ref, w_ref, b_ref, o_ref):
    t = pl.program_id(1)
    kh = t // 4
    kw = t - 4 * kh
    m = (kh + 1) // 2 - 1
    n = (kw + 1) // 2 - 1
    op = _shift_mask(x_ref[0, 0], m, n, w_, nrow)
    acc = _mm(w_ref[0], op)

    @pl.when(t == 0)
    def _():
        o_ref[0] = acc

    @pl.when(t > 0)
    def _():
        o_ref[0] = o_ref[0] + acc

    @pl.when(t == 15)
    def _():
        o_ref[0] = o_ref[0] + b_ref[...]


def _down_conv(x, W, b):
    B, C, H, Wd = x.shape
    h, w = H // 2, Wd // 2
    N = h * w
    pln = (x.reshape(B, C, h, 2, w, 2).transpose(0, 3, 5, 1, 2, 4)
           .reshape(B, 4, C, N))
    Wt = W.transpose(2, 3, 0, 1).reshape(16, C, C)
    b2 = b.reshape(C, 1)

    def xmap(b_, t):
        kh, kw = t // 4, t % 4
        return (b_, ((kh + 1) % 2) * 2 + (kw + 1) % 2, 0, 0)

    return pl.pallas_call(
        functools.partial(_down_body, w, h),
        grid=(B, 16),
        in_specs=[
            pl.BlockSpec((1, 1, C, N), xmap),
            pl.BlockSpec((1, C, C), lambda b_, t: (t, 0, 0)),
            pl.BlockSpec((C, 1), lambda b_, t: (0, 0)),
        ],
        out_specs=pl.BlockSpec((1, C, N), lambda b_, t: (b_, 0, 0)),
        out_shape=jax.ShapeDtypeStruct((B, C, N), F32),
    )(pln, Wt, b2)


def _coarse_body(kf, xd_ref, w_ref, b_ref, o_ref, sel_ref):
    tok = xd_ref[0]
    N = tok.shape[1]
    scale = HD ** (-0.5)
    t = tok.T
    qkv = _mm(t, w_ref[...]) + b_ref[...]
    q = qkv[:, :HD]
    k = qkv[:, HD:2 * HD]
    v = qkv[:, 2 * HD:]
    ls = _dot(q, k, ((1,), (1,))) * scale
    m = jnp.max(ls, axis=1, keepdims=True)
    p = jnp.exp(ls - m)
    s = jnp.sum(p, axis=1, keepdims=True)
    attn = p / s
    score = jnp.sum(attn, axis=0, keepdims=True)
    out = _mm(attn, v)
    o_ref[0] = out.T

    bits = lax.bitcast_convert_type(score, jnp.int32)

    def body(i, th):
        cand = th | lax.shift_left(jnp.int32(1), 30 - i)
        cnt = jnp.sum((bits >= cand).astype(jnp.int32))
        return jnp.where(cnt >= kf, cand, th)

    th = lax.fori_loop(0, 31, body, jnp.int32(0))
    gt = bits > th
    eq = bits == th
    n_gt = jnp.sum(gt.astype(jnp.int32))
    deficit = (kf - n_gt).astype(F32)
    r = lax.broadcasted_iota(jnp.int32, (N, N), 0)
    c = lax.broadcasted_iota(jnp.int32, (N, N), 1)
    ut = (r <= c).astype(F32)
    cum_eq = _mm(eq.astype(F32), ut)
    mask = jnp.logical_or(gt, jnp.logical_and(eq, cum_eq <= deficit))
    slot = (_mm(mask.astype(F32), ut) - 1.0).astype(jnp.int32)
    si = lax.broadcasted_iota(jnp.int32, (N, kf), 1)
    oh = jnp.logical_and(slot.T == si, mask.T).astype(F32)
    sel_ref[0, 0] = oh


def _coarse_attn(xd, cW, cb, kf):
    B, C, N = xd.shape
    nh = C // HD
    cb2 = cb.reshape(1, 3 * HD)
    return pl.pallas_call(
        functools.partial(_coarse_body, kf),
        grid=(B, nh),
        in_specs=[
            pl.BlockSpec((1, HD, N), lambda b_, h_: (b_, h_, 0)),
            pl.BlockSpec((HD, 3 * HD), lambda b_, h_: (0, 0)),
            pl.BlockSpec((1, 3 * HD), lambda b_, h_: (0, 0)),
        ],
        out_specs=(
            pl.BlockSpec((1, HD, N), lambda b_, h_: (b_, h_, 0)),
            pl.BlockSpec((1, 1, N, kf), lambda b_, h_: (b_, h_, 0, 0)),
        ),
        out_shape=(
            jax.ShapeDtypeStruct((B, C, N), F32),
            jax.ShapeDtypeStruct((B, nh, N, kf), F32),
        ),
    )(xd, cW, cb2)


def _up_body(w_, nrow, x_ref, w_ref, b_ref, o_ref):
    f = pl.program_id(1)
    j = pl.program_id(2)
    m = f // 2 - j // 2
    n = f % 2 - j % 2
    op = _shift_mask(x_ref[0], m, n, w_, nrow)
    acc = _mm(w_ref[0, 0], op)

    @pl.when(j == 0)
    def _():
        o_ref[0, 0] = acc

    @pl.when(j > 0)
    def _():
        o_ref[0, 0] = o_ref[0, 0] + acc

    @pl.when(j == 3)
    def _():
        o_ref[0, 0] = o_ref[0, 0] + b_ref[...]


def _up_conv(x, W, b, h, w):
    B, C, N = x.shape
    Wt = W.transpose(2, 3, 1, 0)
    Wup = jnp.stack([
        jnp.stack([Wt[2 * jh + (1 - ei), 2 * jw + (1 - ej)]
                   for jh in range(2) for jw in range(2)], axis=0)
        for ei in range(2) for ej in range(2)], axis=0)
    b2 = b.reshape(C, 1)
    return pl.pallas_call(
        functools.partial(_up_body, w, h),
        grid=(B, 4, 4),
        in_specs=[
            pl.BlockSpec((1, C, N), lambda b_, f, j: (b_, 0, 0)),
            pl.BlockSpec((1, 1, C, C), lambda b_, f, j: (f, j, 0, 0)),
            pl.BlockSpec((C, 1), lambda b_, f, j: (0, 0)),
        ],
        out_specs=pl.BlockSpec((1, 1, C, N), lambda b_, f, j: (b_, f, 0, 0)),
        out_shape=jax.ShapeDtypeStruct((B, 4, C, N), F32),
    )(x, Wup, b2)


def _topk_body(co_ref, sel_ref, w_ref, b_ref, o_ref):
    sel = sel_ref[0, 0]
    cp = co_ref[0]
    kf = sel.shape[1]
    scale = HD ** (-0.5)
    tsel = [_dot(sel, cp[f], ((0,), (1,))) for f in range(4)]
    toks = jnp.concatenate(tsel, axis=0)
    qkv = _mm(toks, w_ref[...]) + b_ref[...]
    q = qkv[:, :HD]
    k = qkv[:, HD:2 * HD]
    v = qkv[:, 2 * HD:]
    ls = _dot(q, k, ((1,), (1,))) * scale
    m = jnp.max(ls, axis=1, keepdims=True)
    p = jnp.exp(ls - m)
    s = jnp.sum(p, axis=1, keepdims=True)
    out = _mm(p / s, v)
    for f in range(4):
        og = out[f * kf:(f + 1) * kf, :]
        scat = _dot(og, sel, ((0,), (1,)))
        o_ref[0, f] = scat + cp[f]


def _topk_attn(co_ph, sel, tW, tb):
    B, _, C, N = co_ph.shape
    nh = C // HD
    kf = sel.shape[3]
    tb2 = tb.reshape(1, 3 * HD)
    return pl.pallas_call(
        _topk_body,
        grid=(B, nh),
        in_specs=[
            pl.BlockSpec((1, 4, HD, N), lambda b_, h_: (b_, 0, h_, 0)),
            pl.BlockSpec((1, 1, N, kf), lambda b_, h_: (b_, h_, 0, 0)),
            pl.BlockSpec((HD, 3 * HD), lambda b_, h_: (0, 0)),
            pl.BlockSpec((1, 3 * HD), lambda b_, h_: (0, 0)),
        ],
        out_specs=pl.BlockSpec((1, 4, HD, N), lambda b_, h_: (b_, 0, h_, 0)),
        out_shape=jax.ShapeDtypeStruct((B, 4, C, N), F32),
    )(co_ph, sel, tW, tb2)


def _dw_body(w_, nrow, x_ref, w_ref, g_ref, be_ref, o_ref):
    xp = x_ref[0]
    cb = xp.shape[1]
    N = xp.shape[2]
    for a in range(2):
        for b2 in range(2):
            y = jnp.zeros((cb, N), F32)
            for u in range(3):
                for v in range(3):
                    ap, m = (a + u - 1) % 2, (a + u - 1) // 2
                    bp, n = (b2 + v - 1) % 2, (b2 + v - 1) // 2
                    src = _shift_mask(xp[ap * 2 + bp], m, n, w_, nrow)
                    y = y + w_ref[u * 3 + v] * src
            y = y * g_ref[...] + be_ref[...]
            o_ref[0, a * 2 + b2] = jnp.clip(y, 0.0, 6.0)


def _dw_conv(x2_ph, dw_W, dw_g, dw_be, h, w):
    B, _, C, N = x2_ph.shape
    cbs = 128 if C % 128 == 0 else C
    nc = C // cbs
    dwr = dw_W.reshape(C, 9).T.reshape(9, C, 1)
    g = dw_g.reshape(C, 1)
    be = dw_be.reshape(C, 1)
    return pl.pallas_call(
        functools.partial(_dw_body, w, h),
        grid=(B, nc),
        in_specs=[
            pl.BlockSpec((1, 4, cbs, N), lambda b_, c_: (b_, 0, c_, 0)),
            pl.BlockSpec((9, cbs, 1), lambda b_, c_: (0, c_, 0)),
            pl.BlockSpec((cbs, 1), lambda b_, c_: (c_, 0)),
            pl.BlockSpec((cbs, 1), lambda b_, c_: (c_, 0)),
        ],
        out_specs=pl.BlockSpec((1, 4, cbs, N), lambda b_, c_: (b_, 0, c_, 0)),
        out_shape=jax.ShapeDtypeStruct((B, 4, C, N), F32),
    )(x2_ph, dwr, g, be)


def _pw_body(x_ref, w_ref, g_ref, be_ref, o_ref):
    z = _mm(w_ref[...], x_ref[0, 0])
    z = z * g_ref[...] + be_ref[...]
    o_ref[0, 0] = jnp.clip(z, 0.0, 6.0)


def _pw_conv(t_ph, pw_W, pw_g, pw_be):
    B, _, C, N = t_ph.shape
    W2 = pw_W.reshape(C, C)
    g = pw_g.reshape(C, 1)
    be = pw_be.reshape(C, 1)
    return pl.pallas_call(
        _pw_body,
        grid=(B, 4),
        in_specs=[
            pl.BlockSpec((1, 1, C, N), lambda b_, f: (b_, f, 0, 0)),
            pl.BlockSpec((C, C), lambda b_, f: (0, 0)),
            pl.BlockSpec((C, 1), lambda b_, f: (0, 0)),
            pl.BlockSpec((C, 1), lambda b_, f: (0, 0)),
        ],
        out_specs=pl.BlockSpec((1, 1, C, N), lambda b_, f: (b_, f, 0, 0)),
        out_shape=jax.ShapeDtypeStruct((B, 4, C, N), F32),
    )(t_ph, W2, g, be)


def kernel(x, down_W, down_b, up_W, up_b, cW, cb, tW, tb,
           dw_W, dw_g, dw_be, pw_W, pw_g, pw_be):
    B, C, H, W = x.shape
    h, w = H // 2, W // 2
    N = h * w
    kf = max(1, N // 4)

    xd = _down_conv(x, down_W, down_b)
    att, sel = _coarse_attn(xd, cW, cb, kf)
    co_ph = _up_conv(att, up_W, up_b, h, w)
    x2_ph = _topk_attn(co_ph, sel, tW, tb)
    t_ph = _dw_conv(x2_ph, dw_W, dw_g, dw_be, h, w)
    z_ph = _pw_conv(t_ph, pw_W, pw_g, pw_be)
    return (z_ph.reshape(B, 2, 2, C, h, w)
            .transpose(0, 3, 4, 1, 5, 2).reshape(B, C, H, W))


# speedup vs baseline: 4.2100x; 1.1214x over previous
import functools

import jax
import jax.numpy as jnp
from jax import lax
from jax.experimental import pallas as pl
from jax.experimental.pallas import tpu as pltpu
from jax.experimental.pallas import tpu_sc as plsc

HD = 64
SC_L = 16
PREC = lax.Precision.DEFAULT
F32 = jnp.float32


def _dot(a, b, dims):
    return lax.dot_general(a, b, (dims, ((), ())), precision=PREC,
                           preferred_element_type=F32)


def _mm(a, b):
    return _dot(a, b, (((a.ndim - 1,), (0,))))


def _shift_mask(x, m, n, w, nrow):
    N = nrow * w
    d = m * w + n
    sh = pltpu.roll(x, (-d) % N, axis=1)
    li = lax.broadcasted_iota(jnp.int32, (1, N), 1)
    col = li - (li // w) * w
    src = li + d
    valid = jnp.logical_and(
        jnp.logical_and(src >= 0, src < N),
        jnp.logical_and(col + n >= 0, col + n < w))
    return jnp.where(valid, sh, 0.0)


def _down_body(w_, nrow, x_ref, w_ref, b_ref, o_ref):
    t = pl.program_id(1)
    kh = t // 4
    kw = t - 4 * kh
    m = (kh + 1) // 2 - 1
    n = (kw + 1) // 2 - 1
    op = _shift_mask(x_ref[0, 0], m, n, w_, nrow)
    acc = _mm(w_ref[0], op)

    @pl.when(t == 0)
    def _():
        o_ref[0] = acc

    @pl.when(t > 0)
    def _():
        o_ref[0] = o_ref[0] + acc

    @pl.when(t == 15)
    def _():
        o_ref[0] = o_ref[0] + b_ref[...]


def _down_conv(x, W, b):
    B, C, H, Wd = x.shape
    h, w = H // 2, Wd // 2
    N = h * w
    pln = (x.reshape(B, C, h, 2, w, 2).transpose(0, 3, 5, 1, 2, 4)
           .reshape(B, 4, C, N))
    Wt = W.transpose(2, 3, 0, 1).reshape(16, C, C)
    b2 = b.reshape(C, 1)

    def xmap(b_, t):
        kh, kw = t // 4, t % 4
        return (b_, ((kh + 1) % 2) * 2 + (kw + 1) % 2, 0, 0)

    return pl.pallas_call(
        functools.partial(_down_body, w, h),
        grid=(B, 16),
        in_specs=[
            pl.BlockSpec((1, 1, C, N), xmap),
            pl.BlockSpec((1, C, C), lambda b_, t: (t, 0, 0)),
            pl.BlockSpec((C, 1), lambda b_, t: (0, 0)),
        ],
        out_specs=pl.BlockSpec((1, C, N), lambda b_, t: (b_, 0, 0)),
        out_shape=jax.ShapeDtypeStruct((B, C, N), F32),
    )(pln, Wt, b2)


def _coarse_body(kf, xd_ref, w_ref, b_ref, o_ref, sel_ref):
    tok = xd_ref[0]
    N = tok.shape[1]
    scale = HD ** (-0.5)
    t = tok.T
    qkv = _mm(t, w_ref[...]) + b_ref[...]
    q = qkv[:, :HD]
    k = qkv[:, HD:2 * HD]
    v = qkv[:, 2 * HD:]
    ls = _dot(q, k, ((1,), (1,))) * scale
    m = jnp.max(ls, axis=1, keepdims=True)
    p = jnp.exp(ls - m)
    s = jnp.sum(p, axis=1, keepdims=True)
    attn = p / s
    score = jnp.sum(attn, axis=0, keepdims=True)
    out = _mm(attn, v)
    o_ref[0] = out.T
    sel_ref[0, 0] = lax.bitcast_convert_type(score, jnp.int32)


def _coarse_attn(xd, cW, cb, kf):
    B, C, N = xd.shape
    nh = C // HD
    cb2 = cb.reshape(1, 3 * HD)
    return pl.pallas_call(
        functools.partial(_coarse_body, kf),
        grid=(B, nh),
        in_specs=[
            pl.BlockSpec((1, HD, N), lambda b_, h_: (b_, h_, 0)),
            pl.BlockSpec((HD, 3 * HD), lambda b_, h_: (0, 0)),
            pl.BlockSpec((1, 3 * HD), lambda b_, h_: (0, 0)),
        ],
        out_specs=(
            pl.BlockSpec((1, HD, N), lambda b_, h_: (b_, h_, 0)),
            pl.BlockSpec((1, 1, 1, N), lambda b_, h_: (b_, h_, 0, 0)),
        ),
        out_shape=(
            jax.ShapeDtypeStruct((B, C, N), F32),
            jax.ShapeDtypeStruct((B, nh, 1, N), jnp.int32),
        ),
    )(xd, cW, cb2)


def _sc_topk_body(kf, R, N, score_ref, mask_ref, vbuf, mbuf, sem):
    cid = lax.axis_index("c")
    sid = lax.axis_index("s")
    tid = cid * 16 + sid
    nchunk = N // SC_L
    zero_v = jnp.zeros((SC_L,), jnp.int32)

    def total_ge(cand):
        cand_v = jnp.full((SC_L,), cand, jnp.int32)

        def body(i, acc):
            m = vbuf[pl.ds(i * SC_L, SC_L)] >= cand_v
            return acc + jnp.where(m, 1, 0)

        acc = lax.fori_loop(0, nchunk, body, zero_v)
        tot = acc[0]
        for l in range(1, SC_L):
            tot = tot + acc[l]
        return tot

    def do_row(r):
        cp = pltpu.make_async_copy(score_ref.at[r], vbuf, sem)
        cp.start()
        cp.wait()

        def bs_body(b, th):
            cand = th | lax.shift_left(jnp.int32(1), 30 - b)
            cnt = total_ge(cand)
            return jnp.where(cnt >= kf, cand, th)

        th = lax.fori_loop(0, 31, bs_body, jnp.int32(0))
        th_v = jnp.full((SC_L,), th, jnp.int32)

        def sel_body(i, carry):
            bits = vbuf[pl.ds(i * SC_L, SC_L)]
            code = jnp.where(bits > th_v, 2, jnp.where(bits == th_v, 1, 0))
            mbuf[pl.ds(i * SC_L, SC_L)] = code
            return carry

        lax.fori_loop(0, nchunk, sel_body, jnp.int32(0))
        cp2 = pltpu.make_async_copy(mbuf, mask_ref.at[r], sem)
        cp2.start()
        cp2.wait()

    for k in range((R + 31) // 32):
        r = tid + 32 * k

        @pl.when(r < R)
        def _():
            do_row(r)


def _sc_topk_mask(score2d, kf):
    R, N = score2d.shape
    return pl.kernel(
        functools.partial(_sc_topk_body, kf, R, N),
        out_type=jax.ShapeDtypeStruct((R, N), jnp.int32),
        mesh=plsc.VectorSubcoreMesh(core_axis_name="c", subcore_axis_name="s"),
        scratch_types=[
            pltpu.VMEM((N,), jnp.int32),
            pltpu.VMEM((N,), jnp.int32),
            pltpu.SemaphoreType.DMA,
        ],
    )(score2d)


def _up_body(w_, nrow, x_ref, w_ref, b_ref, o_ref):
    f = pl.program_id(1)
    j = pl.program_id(2)
    m = f // 2 - j // 2
    n = f % 2 - j % 2
    op = _shift_mask(x_ref[0], m, n, w_, nrow)
    acc = _mm(w_ref[0, 0], op)

    @pl.when(j == 0)
    def _():
        o_ref[0, 0] = acc

    @pl.when(j > 0)
    def _():
        o_ref[0, 0] = o_ref[0, 0] + acc

    @pl.when(j == 3)
    def _():
        o_ref[0, 0] = o_ref[0, 0] + b_ref[...]


def _up_conv(x, W, b, h, w):
    B, C, N = x.shape
    Wt = W.transpose(2, 3, 1, 0)
    Wup = jnp.stack([
        jnp.stack([Wt[2 * jh + (1 - ei), 2 * jw + (1 - ej)]
                   for jh in range(2) for jw in range(2)], axis=0)
        for ei in range(2) for ej in range(2)], axis=0)
    b2 = b.reshape(C, 1)
    return pl.pallas_call(
        functools.partial(_up_body, w, h),
        grid=(B, 4, 4),
        in_specs=[
            pl.BlockSpec((1, C, N), lambda b_, f, j: (b_, 0, 0)),
            pl.BlockSpec((1, 1, C, C), lambda b_, f, j: (f, j, 0, 0)),
            pl.BlockSpec((C, 1), lambda b_, f, j: (0, 0)),
        ],
        out_specs=pl.BlockSpec((1, 1, C, N), lambda b_, f, j: (b_, f, 0, 0)),
        out_shape=jax.ShapeDtypeStruct((B, 4, C, N), F32),
    )(x, Wup, b2)


def _topk_body(kf, co_ref, msk_ref, w_ref, b_ref, o_ref):
    code = msk_ref[0, 0]
    cp = co_ref[0]
    N = cp.shape[2]
    scale = HD ** (-0.5)
    gt = code == 2
    eq = code == 1
    n_gt = jnp.sum(gt.astype(jnp.int32))
    deficit = (kf - n_gt).astype(F32)
    r = lax.broadcasted_iota(jnp.int32, (N, N), 0)
    c = lax.broadcasted_iota(jnp.int32, (N, N), 1)
    ut = (r <= c).astype(F32)
    cum_eq = _mm(eq.astype(F32), ut)
    msk = jnp.logical_or(gt, jnp.logical_and(eq, cum_eq <= deficit))
    slot = (_mm(msk.astype(F32), ut) - 1.0).astype(jnp.int32)
    si = lax.broadcasted_iota(jnp.int32, (N, kf), 1)
    sel = jnp.logical_and(slot.T == si, msk.T).astype(F32)
    tsel = [_dot(sel, cp[f], ((0,), (1,))) for f in range(4)]
    toks = jnp.concatenate(tsel, axis=0)
    qkv = _mm(toks, w_ref[...]) + b_ref[...]
    q = qkv[:, :HD]
    k = qkv[:, HD:2 * HD]
    v = qkv[:, 2 * HD:]
    ls = _dot(q, k, ((1,), (1,))) * scale
    m = jnp.max(ls, axis=1, keepdims=True)
    p = jnp.exp(ls - m)
    s = jnp.sum(p, axis=1, keepdims=True)
    out = _mm(p / s, v)
    for f in range(4):
        og = out[f * kf:(f + 1) * kf, :]
        scat = _dot(og, sel, ((0,), (1,)))
        o_ref[0, f] = scat + cp[f]


def _topk_attn(co_ph, msk, kf, tW, tb):
    B, _, C, N = co_ph.shape
    nh = C // HD
    tb2 = tb.reshape(1, 3 * HD)
    return pl.pallas_call(
        functools.partial(_topk_body, kf),
        grid=(B, nh),
        in_specs=[
            pl.BlockSpec((1, 4, HD, N), lambda b_, h_: (b_, 0, h_, 0)),
            pl.BlockSpec((1, 1, 1, N), lambda b_, h_: (b_, h_, 0, 0)),
            pl.BlockSpec((HD, 3 * HD), lambda b_, h_: (0, 0)),
            pl.BlockSpec((1, 3 * HD), lambda b_, h_: (0, 0)),
        ],
        out_specs=pl.BlockSpec((1, 4, HD, N), lambda b_, h_: (b_, 0, h_, 0)),
        out_shape=jax.ShapeDtypeStruct((B, 4, C, N), F32),
    )(co_ph, msk, tW, tb2)


def _dw_body(w_, nrow, x_ref, w_ref, g_ref, be_ref, o_ref):
    xp = x_ref[0]
    cb = xp.shape[1]
    N = xp.shape[2]
    for a in range(2):
        for b2 in range(2):
            y = jnp.zeros((cb, N), F32)
            for u in range(3):
                for v in range(3):
                    ap, m = (a + u - 1) % 2, (a + u - 1) // 2
                    bp, n = (b2 + v - 1) % 2, (b2 + v - 1) // 2
                    src = _shift_mask(xp[ap * 2 + bp], m, n, w_, nrow)
                    y = y + w_ref[u * 3 + v] * src
            y = y * g_ref[...] + be_ref[...]
            o_ref[0, a * 2 + b2] = jnp.clip(y, 0.0, 6.0)


def _dw_conv(x2_ph, dw_W, dw_g, dw_be, h, w):
    B, _, C, N = x2_ph.shape
    cbs = 128 if C % 128 == 0 else C
    nc = C // cbs
    dwr = dw_W.reshape(C, 9).T.reshape(9, C, 1)
    g = dw_g.reshape(C, 1)
    be = dw_be.reshape(C, 1)
    return pl.pallas_call(
        functools.partial(_dw_body, w, h),
        grid=(B, nc),
        in_specs=[
            pl.BlockSpec((1, 4, cbs, N), lambda b_, c_: (b_, 0, c_, 0)),
            pl.BlockSpec((9, cbs, 1), lambda b_, c_: (0, c_, 0)),
            pl.BlockSpec((cbs, 1), lambda b_, c_: (c_, 0)),
            pl.BlockSpec((cbs, 1), lambda b_, c_: (c_, 0)),
        ],
        out_specs=pl.BlockSpec((1, 4, cbs, N), lambda b_, c_: (b_, 0, c_, 0)),
        out_shape=jax.ShapeDtypeStruct((B, 4, C, N), F32),
    )(x2_ph, dwr, g, be)


def _pw_body(x_ref, w_ref, g_ref, be_ref, o_ref):
    z = _mm(w_ref[...], x_ref[0, 0])
    z = z * g_ref[...] + be_ref[...]
    o_ref[0, 0] = jnp.clip(z, 0.0, 6.0)


def _pw_conv(t_ph, pw_W, pw_g, pw_be):
    B, _, C, N = t_ph.shape
    W2 = pw_W.reshape(C, C)
    g = pw_g.reshape(C, 1)
    be = pw_be.reshape(C, 1)
    return pl.pallas_call(
        _pw_body,
        grid=(B, 4),
        in_specs=[
            pl.BlockSpec((1, 1, C, N), lambda b_, f: (b_, f, 0, 0)),
            pl.BlockSpec((C, C), lambda b_, f: (0, 0)),
            pl.BlockSpec((C, 1), lambda b_, f: (0, 0)),
            pl.BlockSpec((C, 1), lambda b_, f: (0, 0)),
        ],
        out_specs=pl.BlockSpec((1, 1, C, N), lambda b_, f: (b_, f, 0, 0)),
        out_shape=jax.ShapeDtypeStruct((B, 4, C, N), F32),
    )(t_ph, W2, g, be)


def kernel(x, down_W, down_b, up_W, up_b, cW, cb, tW, tb,
           dw_W, dw_g, dw_be, pw_W, pw_g, pw_be):
    B, C, H, W = x.shape
    h, w = H // 2, W // 2
    N = h * w
    kf = max(1, N // 4)

    nh = C // HD
    xd = _down_conv(x, down_W, down_b)
    att, score = _coarse_attn(xd, cW, cb, kf)
    msk = _sc_topk_mask(score.reshape(B * nh, N), kf)
    co_ph = _up_conv(att, up_W, up_b, h, w)
    x2_ph = _topk_attn(co_ph, msk.reshape(B, nh, 1, N), kf, tW, tb)
    t_ph = _dw_conv(x2_ph, dw_W, dw_g, dw_be, h, w)
    z_ph = _pw_conv(t_ph, pw_W, pw_g, pw_be)
    return (z_ph.reshape(B, 2, 2, C, h, w)
            .transpose(0, 3, 4, 1, 5, 2).reshape(B, C, H, W))


# speedup vs baseline: 4.2969x; 1.0206x over previous
import functools

import jax
import jax.numpy as jnp
from jax import lax
from jax.experimental import pallas as pl
from jax.experimental.pallas import tpu as pltpu
from jax.experimental.pallas import tpu_sc as plsc

HD = 64
SC_L = 16
PREC = lax.Precision.DEFAULT
F32 = jnp.float32


def _dot(a, b, dims):
    return lax.dot_general(a, b, (dims, ((), ())), precision=PREC,
                           preferred_element_type=F32)


def _mm(a, b):
    return _dot(a, b, (((a.ndim - 1,), (0,))))


def _shift_mask(x, m, n, w, nrow):
    N = nrow * w
    d = m * w + n
    sh = pltpu.roll(x, (-d) % N, axis=1)
    li = lax.broadcasted_iota(jnp.int32, (1, N), 1)
    col = li - (li // w) * w
    src = li + d
    valid = jnp.logical_and(
        jnp.logical_and(src >= 0, src < N),
        jnp.logical_and(col + n >= 0, col + n < w))
    return jnp.where(valid, sh, 0.0)


def _down_body(w_, nrow, x_ref, w_ref, b_ref, o_ref):
    t = pl.program_id(1)
    pp, i = t // 4, t % 4
    a, b2 = pp // 2, pp % 2
    kh = (1 - a) + 2 * (i // 2)
    kw = (1 - b2) + 2 * (i % 2)
    m = (kh + 1) // 2 - 1
    n = (kw + 1) // 2 - 1
    op = _shift_mask(x_ref[0, 0], m, n, w_, nrow)
    acc = _mm(w_ref[0], op)

    @pl.when(t == 0)
    def _():
        o_ref[0] = acc

    @pl.when(t > 0)
    def _():
        o_ref[0] = o_ref[0] + acc

    @pl.when(t == 15)
    def _():
        o_ref[0] = o_ref[0] + b_ref[...]


def _wmap_down(b_, t):
    pp, i = t // 4, t % 4
    a, b2 = pp // 2, pp % 2
    kh = (1 - a) + 2 * (i // 2)
    kw = (1 - b2) + 2 * (i % 2)
    return (kh * 4 + kw, 0, 0)


def _down_conv(x, W, b):
    B, C, H, Wd = x.shape
    h, w = H // 2, Wd // 2
    N = h * w
    pln = (x.reshape(B, C, h, 2, w, 2).transpose(0, 3, 5, 1, 2, 4)
           .reshape(B, 4, C, N))
    Wt = W.transpose(2, 3, 0, 1).reshape(16, C, C)
    b2 = b.reshape(C, 1)

    return pl.pallas_call(
        functools.partial(_down_body, w, h),
        grid=(B, 16),
        in_specs=[
            pl.BlockSpec((1, 1, C, N), lambda b_, t: (b_, t // 4, 0, 0)),
            pl.BlockSpec((1, C, C), _wmap_down),
            pl.BlockSpec((C, 1), lambda b_, t: (0, 0)),
        ],
        out_specs=pl.BlockSpec((1, C, N), lambda b_, t: (b_, 0, 0)),
        out_shape=jax.ShapeDtypeStruct((B, C, N), F32),
    )(pln, Wt, b2)


def _coarse_body(kf, xd_ref, w_ref, b_ref, o_ref, sel_ref):
    tok = xd_ref[0]
    N = tok.shape[1]
    scale = HD ** (-0.5)
    t = tok.T
    qkv = _mm(t, w_ref[...]) + b_ref[...]
    q = qkv[:, :HD]
    k = qkv[:, HD:2 * HD]
    v = qkv[:, 2 * HD:]
    ls = _dot(q, k, ((1,), (1,))) * scale
    m = jnp.max(ls, axis=1, keepdims=True)
    p = jnp.exp(ls - m)
    s = jnp.sum(p, axis=1, keepdims=True)
    attn = p / s
    score = jnp.sum(attn, axis=0, keepdims=True)
    out = _mm(attn, v)
    o_ref[0] = out.T
    sel_ref[0, 0] = lax.bitcast_convert_type(score, jnp.int32)


def _coarse_attn(xd, cW, cb, kf):
    B, C, N = xd.shape
    nh = C // HD
    cb2 = cb.reshape(1, 3 * HD)
    return pl.pallas_call(
        functools.partial(_coarse_body, kf),
        grid=(B, nh),
        in_specs=[
            pl.BlockSpec((1, HD, N), lambda b_, h_: (b_, h_, 0)),
            pl.BlockSpec((HD, 3 * HD), lambda b_, h_: (0, 0)),
            pl.BlockSpec((1, 3 * HD), lambda b_, h_: (0, 0)),
        ],
        out_specs=(
            pl.BlockSpec((1, HD, N), lambda b_, h_: (b_, h_, 0)),
            pl.BlockSpec((1, 1, 1, N), lambda b_, h_: (b_, h_, 0, 0)),
        ),
        out_shape=(
            jax.ShapeDtypeStruct((B, C, N), F32),
            jax.ShapeDtypeStruct((B, nh, 1, N), jnp.int32),
        ),
    )(xd, cW, cb2)


def _sc_topk_body(kf, R, N, score_ref, mask_ref, vbuf, mbuf, sem):
    cid = lax.axis_index("c")
    sid = lax.axis_index("s")
    tid = cid * 16 + sid
    nchunk = N // SC_L
    zero_v = jnp.zeros((SC_L,), jnp.int32)

    def total_ge(cand):
        cand_v = jnp.full((SC_L,), cand, jnp.int32)

        def body(i, acc):
            m = vbuf[pl.ds(i * SC_L, SC_L)] >= cand_v
            return acc + jnp.where(m, 1, 0)

        acc = lax.fori_loop(0, nchunk, body, zero_v)
        tot = acc[0]
        for l in range(1, SC_L):
            tot = tot + acc[l]
        return tot

    def do_row(r):
        cp = pltpu.make_async_copy(score_ref.at[r], vbuf, sem)
        cp.start()
        cp.wait()

        def bs_body(b, th):
            cand = th | lax.shift_left(jnp.int32(1), 30 - b)
            cnt = total_ge(cand)
            return jnp.where(cnt >= kf, cand, th)

        th = lax.fori_loop(0, 31, bs_body, jnp.int32(0))
        th_v = jnp.full((SC_L,), th, jnp.int32)

        def sel_body(i, carry):
            bits = vbuf[pl.ds(i * SC_L, SC_L)]
            code = jnp.where(bits > th_v, 2, jnp.where(bits == th_v, 1, 0))
            mbuf[pl.ds(i * SC_L, SC_L)] = code
            return carry

        lax.fori_loop(0, nchunk, sel_body, jnp.int32(0))
        cp2 = pltpu.make_async_copy(mbuf, mask_ref.at[r], sem)
        cp2.start()
        cp2.wait()

    for k in range((R + 31) // 32):
        r = tid + 32 * k

        @pl.when(r < R)
        def _():
            do_row(r)


def _sc_topk_mask(score2d, kf):
    R, N = score2d.shape
    return pl.kernel(
        functools.partial(_sc_topk_body, kf, R, N),
        out_type=jax.ShapeDtypeStruct((R, N), jnp.int32),
        mesh=plsc.VectorSubcoreMesh(core_axis_name="c", subcore_axis_name="s"),
        scratch_types=[
            pltpu.VMEM((N,), jnp.int32),
            pltpu.VMEM((N,), jnp.int32),
            pltpu.SemaphoreType.DMA,
        ],
    )(score2d)


def _up_body(w_, nrow, x_ref, w_ref, b_ref, o_ref):
    f = pl.program_id(1)
    j = pl.program_id(2)
    m = f // 2 - j // 2
    n = f % 2 - j % 2
    op = _shift_mask(x_ref[0], m, n, w_, nrow)
    acc = _mm(w_ref[0, 0], op.astype(jnp.bfloat16))

    @pl.when(j == 0)
    def _():
        o_ref[0, 0] = acc

    @pl.when(j > 0)
    def _():
        o_ref[0, 0] = o_ref[0, 0] + acc

    @pl.when(j == 3)
    def _():
        o_ref[0, 0] = o_ref[0, 0] + b_ref[...]


def _up_conv(x, W, b, h, w):
    B, C, N = x.shape
    Wt = W.transpose(2, 3, 1, 0)
    Wup = jnp.stack([
        jnp.stack([Wt[2 * jh + (1 - ei), 2 * jw + (1 - ej)]
                   for jh in range(2) for jw in range(2)], axis=0)
        for ei in range(2) for ej in range(2)], axis=0)
    Wup = Wup.astype(jnp.bfloat16)
    b2 = b.reshape(C, 1)
    return pl.pallas_call(
        functools.partial(_up_body, w, h),
        grid=(B, 4, 4),
        in_specs=[
            pl.BlockSpec((1, C, N), lambda b_, f, j: (b_, 0, 0)),
            pl.BlockSpec((1, 1, C, C), lambda b_, f, j: (f, j, 0, 0)),
            pl.BlockSpec((C, 1), lambda b_, f, j: (0, 0)),
        ],
        out_specs=pl.BlockSpec((1, 1, C, N), lambda b_, f, j: (b_, f, 0, 0)),
        out_shape=jax.ShapeDtypeStruct((B, 4, C, N), F32),
    )(x, Wup, b2)


def _topk_body(kf, co_ref, msk_ref, w_ref, b_ref, o_ref):
    code = msk_ref[0, 0]
    cp = co_ref[0]
    N = cp.shape[2]
    scale = HD ** (-0.5)
    gt = code == 2
    eq = code == 1
    n_gt = jnp.sum(gt.astype(jnp.int32))
    deficit = (kf - n_gt).astype(F32)
    r = lax.broadcasted_iota(jnp.int32, (N, N), 0)
    c = lax.broadcasted_iota(jnp.int32, (N, N), 1)
    ut = (r <= c).astype(F32)
    cum_eq = _mm(eq.astype(F32), ut)
    msk = jnp.logical_or(gt, jnp.logical_and(eq, cum_eq <= deficit))
    slot = (_mm(msk.astype(F32), ut) - 1.0).astype(jnp.int32)
    si = lax.broadcasted_iota(jnp.int32, (N, kf), 1)
    sel = jnp.logical_and(slot.T == si, msk.T).astype(F32)
    tsel = [_dot(sel, cp[f], ((0,), (1,))) for f in range(4)]
    toks = jnp.concatenate(tsel, axis=0)
    qkv = _mm(toks, w_ref[...]) + b_ref[...]
    q = qkv[:, :HD]
    k = qkv[:, HD:2 * HD]
    v = qkv[:, 2 * HD:]
    ls = _dot(q, k, ((1,), (1,))) * scale
    m = jnp.max(ls, axis=1, keepdims=True)
    p = jnp.exp(ls - m)
    s = jnp.sum(p, axis=1, keepdims=True)
    out = _mm(p / s, v)
    for f in range(4):
        og = out[f * kf:(f + 1) * kf, :]
        scat = _dot(og, sel, ((0,), (1,)))
        o_ref[0, f] = scat + cp[f]


def _topk_attn(co_ph, msk, kf, tW, tb):
    B, _, C, N = co_ph.shape
    nh = C // HD
    tb2 = tb.reshape(1, 3 * HD)
    return pl.pallas_call(
        functools.partial(_topk_body, kf),
        grid=(B, nh),
        in_specs=[
            pl.BlockSpec((1, 4, HD, N), lambda b_, h_: (b_, 0, h_, 0)),
            pl.BlockSpec((1, 1, 1, N), lambda b_, h_: (b_, h_, 0, 0)),
            pl.BlockSpec((HD, 3 * HD), lambda b_, h_: (0, 0)),
            pl.BlockSpec((1, 3 * HD), lambda b_, h_: (0, 0)),
        ],
        out_specs=pl.BlockSpec((1, 4, HD, N), lambda b_, h_: (b_, 0, h_, 0)),
        out_shape=jax.ShapeDtypeStruct((B, 4, C, N), F32),
    )(co_ph, msk, tW, tb2)


def _dw_body(w_, nrow, x_ref, w_ref, g_ref, be_ref, o_ref):
    xp = x_ref[0]
    cb = xp.shape[1]
    N = xp.shape[2]
    for a in range(2):
        for b2 in range(2):
            y = jnp.zeros((cb, N), F32)
            for u in range(3):
                for v in range(3):
                    ap, m = (a + u - 1) % 2, (a + u - 1) // 2
                    bp, n = (b2 + v - 1) % 2, (b2 + v - 1) // 2
                    src = _shift_mask(xp[ap * 2 + bp], m, n, w_, nrow)
                    y = y + w_ref[u * 3 + v] * src
            y = y * g_ref[...] + be_ref[...]
            o_ref[0, a * 2 + b2] = jnp.clip(y, 0.0, 6.0)


def _dw_conv(x2_ph, dw_W, dw_g, dw_be, h, w):
    B, _, C, N = x2_ph.shape
    cbs = 128 if C % 128 == 0 else C
    nc = C // cbs
    dwr = dw_W.reshape(C, 9).T.reshape(9, C, 1)
    g = dw_g.reshape(C, 1)
    be = dw_be.reshape(C, 1)
    return pl.pallas_call(
        functools.partial(_dw_body, w, h),
        grid=(B, nc),
        in_specs=[
            pl.BlockSpec((1, 4, cbs, N), lambda b_, c_: (b_, 0, c_, 0)),
            pl.BlockSpec((9, cbs, 1), lambda b_, c_: (0, c_, 0)),
            pl.BlockSpec((cbs, 1), lambda b_, c_: (c_, 0)),
            pl.BlockSpec((cbs, 1), lambda b_, c_: (c_, 0)),
        ],
        out_specs=pl.BlockSpec((1, 4, cbs, N), lambda b_, c_: (b_, 0, c_, 0)),
        out_shape=jax.ShapeDtypeStruct((B, 4, C, N), F32),
    )(x2_ph, dwr, g, be)


def _pw_body(x_ref, w_ref, g_ref, be_ref, o_ref):
    z = _mm(w_ref[...], x_ref[0, 0])
    z = z * g_ref[...] + be_ref[...]
    o_ref[0, 0] = jnp.clip(z, 0.0, 6.0)


def _pw_conv(t_ph, pw_W, pw_g, pw_be):
    B, _, C, N = t_ph.shape
    W2 = pw_W.reshape(C, C)
    g = pw_g.reshape(C, 1)
    be = pw_be.reshape(C, 1)
    return pl.pallas_call(
        _pw_body,
        grid=(B, 4),
        in_specs=[
            pl.BlockSpec((1, 1, C, N), lambda b_, f: (b_, f, 0, 0)),
            pl.BlockSpec((C, C), lambda b_, f: (0, 0)),
            pl.BlockSpec((C, 1), lambda b_, f: (0, 0)),
            pl.BlockSpec((C, 1), lambda b_, f: (0, 0)),
        ],
        out_specs=pl.BlockSpec((1, 1, C, N), lambda b_, f: (b_, f, 0, 0)),
        out_shape=jax.ShapeDtypeStruct((B, 4, C, N), F32),
    )(t_ph, W2, g, be)


def kernel(x, down_W, down_b, up_W, up_b, cW, cb, tW, tb,
           dw_W, dw_g, dw_be, pw_W, pw_g, pw_be):
    B, C, H, W = x.shape
    h, w = H // 2, W // 2
    N = h * w
    kf = max(1, N // 4)

    nh = C // HD
    xd = _down_conv(x, down_W, down_b)
    att, score = _coarse_attn(xd, cW, cb, kf)
    msk = _sc_topk_mask(score.reshape(B * nh, N), kf)
    co_ph = _up_conv(att, up_W, up_b, h, w)
    x2_ph = _topk_attn(co_ph, msk.reshape(B, nh, 1, N), kf, tW, tb)
    t_ph = _dw_conv(x2_ph, dw_W, dw_g, dw_be, h, w)
    z_ph = _pw_conv(t_ph, pw_W, pw_g, pw_be)
    return (z_ph.reshape(B, 2, 2, C, h, w)
            .transpose(0, 3, 4, 1, 5, 2).reshape(B, C, H, W))


# speedup vs baseline: 4.4143x; 1.0273x over previous
import functools

import jax
import jax.numpy as jnp
from jax import lax
from jax.experimental import pallas as pl
from jax.experimental.pallas import tpu as pltpu
from jax.experimental.pallas import tpu_sc as plsc

HD = 64
SC_L = 16
PREC = lax.Precision.DEFAULT
F32 = jnp.float32


def _dot(a, b, dims):
    return lax.dot_general(a, b, (dims, ((), ())), precision=PREC,
                           preferred_element_type=F32)


def _mm(a, b):
    return _dot(a, b, (((a.ndim - 1,), (0,))))


def _shift_mask(x, m, n, w, nrow):
    N = nrow * w
    d = m * w + n
    sh = pltpu.roll(x, (-d) % N, axis=1)
    li = lax.broadcasted_iota(jnp.int32, (1, N), 1)
    col = li - (li // w) * w
    src = li + d
    valid = jnp.logical_and(
        jnp.logical_and(src >= 0, src < N),
        jnp.logical_and(col + n >= 0, col + n < w))
    return jnp.where(valid, sh, 0.0)


def _down_body(w_, nrow, x_ref, w_ref, b_ref, o_ref):
    t = pl.program_id(1)
    pp, i = t // 4, t % 4
    a, b2 = pp // 2, pp % 2
    kh = (1 - a) + 2 * (i // 2)
    kw = (1 - b2) + 2 * (i % 2)
    m = (kh + 1) // 2 - 1
    n = (kw + 1) // 2 - 1
    op = _shift_mask(x_ref[0, 0], m, n, w_, nrow)
    acc = _mm(w_ref[0], op)

    @pl.when(t == 0)
    def _():
        o_ref[0] = acc

    @pl.when(t > 0)
    def _():
        o_ref[0] = o_ref[0] + acc

    @pl.when(t == 15)
    def _():
        o_ref[0] = o_ref[0] + b_ref[...]


def _wmap_down(b_, t):
    pp, i = t // 4, t % 4
    a, b2 = pp // 2, pp % 2
    kh = (1 - a) + 2 * (i // 2)
    kw = (1 - b2) + 2 * (i % 2)
    return (kh * 4 + kw, 0, 0)


def _down_conv(x, W, b):
    B, C, H, Wd = x.shape
    h, w = H // 2, Wd // 2
    N = h * w
    pln = (x.reshape(B, C, h, 2, w, 2).transpose(0, 3, 5, 1, 2, 4)
           .reshape(B, 4, C, N))
    Wt = W.transpose(2, 3, 0, 1).reshape(16, C, C)
    b2 = b.reshape(C, 1)

    return pl.pallas_call(
        functools.partial(_down_body, w, h),
        grid=(B, 16),
        in_specs=[
            pl.BlockSpec((1, 1, C, N), lambda b_, t: (b_, t // 4, 0, 0)),
            pl.BlockSpec((1, C, C), _wmap_down),
            pl.BlockSpec((C, 1), lambda b_, t: (0, 0)),
        ],
        out_specs=pl.BlockSpec((1, C, N), lambda b_, t: (b_, 0, 0)),
        out_shape=jax.ShapeDtypeStruct((B, C, N), F32),
    )(pln, Wt, b2)


def _coarse_body(kf, xd_ref, w_ref, b_ref, o_ref, sel_ref):
    tok = xd_ref[0]
    N = tok.shape[1]
    scale = HD ** (-0.5)
    t = tok.T
    qkv = _mm(t, w_ref[...]) + b_ref[...]
    q = qkv[:, :HD]
    k = qkv[:, HD:2 * HD]
    v = qkv[:, 2 * HD:]
    ls = _dot(q, k, ((1,), (1,))) * scale
    m = jnp.max(ls, axis=1, keepdims=True)
    p = jnp.exp(ls - m)
    s = jnp.sum(p, axis=1, keepdims=True)
    attn = p / s
    score = jnp.sum(attn, axis=0, keepdims=True)
    out = _mm(attn, v)
    o_ref[0] = out.T
    sel_ref[0, 0] = lax.bitcast_convert_type(score, jnp.int32)


def _coarse_attn(xd, cW, cb, kf):
    B, C, N = xd.shape
    nh = C // HD
    cb2 = cb.reshape(1, 3 * HD)
    return pl.pallas_call(
        functools.partial(_coarse_body, kf),
        grid=(B, nh),
        in_specs=[
            pl.BlockSpec((1, HD, N), lambda b_, h_: (b_, h_, 0)),
            pl.BlockSpec((HD, 3 * HD), lambda b_, h_: (0, 0)),
            pl.BlockSpec((1, 3 * HD), lambda b_, h_: (0, 0)),
        ],
        out_specs=(
            pl.BlockSpec((1, HD, N), lambda b_, h_: (b_, h_, 0)),
            pl.BlockSpec((1, 1, 1, N), lambda b_, h_: (b_, h_, 0, 0)),
        ),
        out_shape=(
            jax.ShapeDtypeStruct((B, C, N), F32),
            jax.ShapeDtypeStruct((B, nh, 1, N), jnp.int32),
        ),
    )(xd, cW, cb2)


def _sc_topk_body(kf, R, N, score_ref, mask_ref, vbuf, mbuf, sem):
    cid = lax.axis_index("c")
    sid = lax.axis_index("s")
    tid = cid * 16 + sid
    nchunk = N // SC_L
    zero_v = jnp.zeros((SC_L,), jnp.int32)

    def total_ge(cand):
        cand_v = jnp.full((SC_L,), cand, jnp.int32)

        def body(i, acc):
            m = vbuf[pl.ds(i * SC_L, SC_L)] >= cand_v
            return acc + jnp.where(m, 1, 0)

        acc = lax.fori_loop(0, nchunk, body, zero_v)
        tot = acc[0]
        for l in range(1, SC_L):
            tot = tot + acc[l]
        return tot

    def do_row(r):
        cp = pltpu.make_async_copy(score_ref.at[r], vbuf, sem)
        cp.start()
        cp.wait()

        def bs_body(b, th):
            cand = th | lax.shift_left(jnp.int32(1), 30 - b)
            cnt = total_ge(cand)
            return jnp.where(cnt >= kf, cand, th)

        th = lax.fori_loop(0, 31, bs_body, jnp.int32(0))
        th_v = jnp.full((SC_L,), th, jnp.int32)

        def sel_body(i, carry):
            bits = vbuf[pl.ds(i * SC_L, SC_L)]
            code = jnp.where(bits > th_v, 2, jnp.where(bits == th_v, 1, 0))
            mbuf[pl.ds(i * SC_L, SC_L)] = code
            return carry

        lax.fori_loop(0, nchunk, sel_body, jnp.int32(0))
        cp2 = pltpu.make_async_copy(mbuf, mask_ref.at[r], sem)
        cp2.start()
        cp2.wait()

    for k in range((R + 31) // 32):
        r = tid + 32 * k

        @pl.when(r < R)
        def _():
            do_row(r)


def _sc_topk_mask(score2d, kf):
    R, N = score2d.shape
    return pl.kernel(
        functools.partial(_sc_topk_body, kf, R, N),
        out_type=jax.ShapeDtypeStruct((R, N), jnp.int32),
        mesh=plsc.VectorSubcoreMesh(core_axis_name="c", subcore_axis_name="s"),
        scratch_types=[
            pltpu.VMEM((N,), jnp.int32),
            pltpu.VMEM((N,), jnp.int32),
            pltpu.SemaphoreType.DMA,
        ],
    )(score2d)


def _up_body(w_, nrow, x_ref, w_ref, b_ref, o_ref):
    f = pl.program_id(1)
    j = pl.program_id(2)
    m = f // 2 - j // 2
    n = f % 2 - j % 2
    op = _shift_mask(x_ref[0], m, n, w_, nrow)
    acc = _mm(w_ref[0, 0], op.astype(jnp.bfloat16))

    @pl.when(j == 0)
    def _():
        o_ref[0, 0] = acc

    @pl.when(j > 0)
    def _():
        o_ref[0, 0] = o_ref[0, 0] + acc

    @pl.when(j == 3)
    def _():
        o_ref[0, 0] = o_ref[0, 0] + b_ref[...]


def _up_conv(x, W, b, h, w):
    B, C, N = x.shape
    Wt = W.transpose(2, 3, 1, 0)
    Wup = jnp.stack([
        jnp.stack([Wt[2 * jh + (1 - ei), 2 * jw + (1 - ej)]
                   for jh in range(2) for jw in range(2)], axis=0)
        for ei in range(2) for ej in range(2)], axis=0)
    Wup = Wup.astype(jnp.bfloat16)
    b2 = b.reshape(C, 1)
    return pl.pallas_call(
        functools.partial(_up_body, w, h),
        grid=(B, 4, 4),
        in_specs=[
            pl.BlockSpec((1, C, N), lambda b_, f, j: (b_, 0, 0)),
            pl.BlockSpec((1, 1, C, C), lambda b_, f, j: (f, j, 0, 0)),
            pl.BlockSpec((C, 1), lambda b_, f, j: (0, 0)),
        ],
        out_specs=pl.BlockSpec((1, 1, C, N), lambda b_, f, j: (b_, f, 0, 0)),
        out_shape=jax.ShapeDtypeStruct((B, 4, C, N), F32),
    )(x, Wup, b2)


def _topk_body(kf, co_ref, msk_ref, w_ref, b_ref, o_ref):
    code = msk_ref[0, 0]
    cp = co_ref[0]
    N = cp.shape[2]
    scale = HD ** (-0.5)
    gt = code == 2
    eq = code == 1
    n_gt = jnp.sum(gt.astype(jnp.int32))
    deficit = (kf - n_gt).astype(F32)
    r = lax.broadcasted_iota(jnp.int32, (N, N), 0)
    c = lax.broadcasted_iota(jnp.int32, (N, N), 1)
    ut = (r <= c).astype(F32)
    cum_eq = _mm(eq.astype(F32), ut)
    msk = jnp.logical_or(gt, jnp.logical_and(eq, cum_eq <= deficit))
    slot = (_mm(msk.astype(F32), ut) - 1.0).astype(jnp.int32)
    si = lax.broadcasted_iota(jnp.int32, (N, kf), 1)
    sel = jnp.logical_and(slot.T == si, msk.T).astype(F32)
    tsel = [_dot(sel, cp[f], ((0,), (1,))) for f in range(4)]
    toks = jnp.concatenate(tsel, axis=0)
    qkv = _mm(toks, w_ref[...]) + b_ref[...]
    q = qkv[:, :HD]
    k = qkv[:, HD:2 * HD]
    v = qkv[:, 2 * HD:]
    ls = _dot(q, k, ((1,), (1,))) * scale
    m = jnp.max(ls, axis=1, keepdims=True)
    p = jnp.exp(ls - m)
    s = jnp.sum(p, axis=1, keepdims=True)
    out = _mm(p / s, v)
    for f in range(4):
        og = out[f * kf:(f + 1) * kf, :]
        scat = _dot(og, sel, ((0,), (1,)))
        o_ref[0, f] = (scat + cp[f]).astype(jnp.bfloat16)


def _topk_attn(co_ph, msk, kf, tW, tb):
    B, _, C, N = co_ph.shape
    nh = C // HD
    tb2 = tb.reshape(1, 3 * HD)
    return pl.pallas_call(
        functools.partial(_topk_body, kf),
        grid=(B, nh),
        in_specs=[
            pl.BlockSpec((1, 4, HD, N), lambda b_, h_: (b_, 0, h_, 0)),
            pl.BlockSpec((1, 1, 1, N), lambda b_, h_: (b_, h_, 0, 0)),
            pl.BlockSpec((HD, 3 * HD), lambda b_, h_: (0, 0)),
            pl.BlockSpec((1, 3 * HD), lambda b_, h_: (0, 0)),
        ],
        out_specs=pl.BlockSpec((1, 4, HD, N), lambda b_, h_: (b_, 0, h_, 0)),
        out_shape=jax.ShapeDtypeStruct((B, 4, C, N), jnp.bfloat16),
    )(co_ph, msk, tW, tb2)


def _dw_body(w_, nrow, x_ref, w_ref, g_ref, be_ref, o_ref):
    xp = x_ref[0]
    cb = xp.shape[1]
    N = xp.shape[2]
    for a in range(2):
        for b2 in range(2):
            y = jnp.zeros((cb, N), F32)
            for u in range(3):
                for v in range(3):
                    ap, m = (a + u - 1) % 2, (a + u - 1) // 2
                    bp, n = (b2 + v - 1) % 2, (b2 + v - 1) // 2
                    src = _shift_mask(xp[ap * 2 + bp].astype(F32),
                                      m, n, w_, nrow)
                    y = y + w_ref[u * 3 + v] * src
            y = y * g_ref[...] + be_ref[...]
            o_ref[0, a * 2 + b2] = jnp.clip(y, 0.0, 6.0).astype(jnp.bfloat16)


def _dw_conv(x2_ph, dw_W, dw_g, dw_be, h, w):
    B, _, C, N = x2_ph.shape
    cbs = 128 if C % 128 == 0 else C
    nc = C // cbs
    dwr = dw_W.reshape(C, 9).T.reshape(9, C, 1)
    g = dw_g.reshape(C, 1)
    be = dw_be.reshape(C, 1)
    return pl.pallas_call(
        functools.partial(_dw_body, w, h),
        grid=(B, nc),
        in_specs=[
            pl.BlockSpec((1, 4, cbs, N), lambda b_, c_: (b_, 0, c_, 0)),
            pl.BlockSpec((9, cbs, 1), lambda b_, c_: (0, c_, 0)),
            pl.BlockSpec((cbs, 1), lambda b_, c_: (c_, 0)),
            pl.BlockSpec((cbs, 1), lambda b_, c_: (c_, 0)),
        ],
        out_specs=pl.BlockSpec((1, 4, cbs, N), lambda b_, c_: (b_, 0, c_, 0)),
        out_shape=jax.ShapeDtypeStruct((B, 4, C, N), jnp.bfloat16),
    )(x2_ph, dwr, g, be)


def _pw_body(x_ref, w_ref, g_ref, be_ref, o_ref):
    z = _mm(w_ref[...], x_ref[0, 0])
    z = z * g_ref[...] + be_ref[...]
    o_ref[0, 0] = jnp.clip(z, 0.0, 6.0)


def _pw_conv(t_ph, pw_W, pw_g, pw_be):
    B, _, C, N = t_ph.shape
    W2 = pw_W.reshape(C, C).astype(jnp.bfloat16)
    g = pw_g.reshape(C, 1)
    be = pw_be.reshape(C, 1)
    return pl.pallas_call(
        _pw_body,
        grid=(B, 4),
        in_specs=[
            pl.BlockSpec((1, 1, C, N), lambda b_, f: (b_, f, 0, 0)),
            pl.BlockSpec((C, C), lambda b_, f: (0, 0)),
            pl.BlockSpec((C, 1), lambda b_, f: (0, 0)),
            pl.BlockSpec((C, 1), lambda b_, f: (0, 0)),
        ],
        out_specs=pl.BlockSpec((1, 1, C, N), lambda b_, f: (b_, f, 0, 0)),
        out_shape=jax.ShapeDtypeStruct((B, 4, C, N), F32),
    )(t_ph, W2, g, be)


def kernel(x, down_W, down_b, up_W, up_b, cW, cb, tW, tb,
           dw_W, dw_g, dw_be, pw_W, pw_g, pw_be):
    B, C, H, W = x.shape
    h, w = H // 2, W // 2
    N = h * w
    kf = max(1, N // 4)

    nh = C // HD
    xd = _down_conv(x, down_W, down_b)
    att, score = _coarse_attn(xd, cW, cb, kf)
    msk = _sc_topk_mask(score.reshape(B * nh, N), kf)
    co_ph = _up_conv(att, up_W, up_b, h, w)
    x2_ph = _topk_attn(co_ph, msk.reshape(B, nh, 1, N), kf, tW, tb)
    t_ph = _dw_conv(x2_ph, dw_W, dw_g, dw_be, h, w)
    z_ph = _pw_conv(t_ph, pw_W, pw_g, pw_be)
    return (z_ph.reshape(B, 2, 2, C, h, w)
            .transpose(0, 3, 4, 1, 5, 2).reshape(B, C, H, W))
